# baseline (speedup 1.0000x reference)
import jax
import jax.numpy as jnp
from jax import lax
from jax.experimental import pallas as pl
from jax.experimental.pallas import tpu as pltpu
from jax.experimental.pallas import tpu_sc as plsc

N = 10000
E = 320000
D = 128
H = 128
C = 40
DH = D // 2

NC = 2
NS = 16
CH = 128
NCHUNK = 157
EPT = NCHUNK * CH
EPAD = NS * EPT - E
NP = 10240
ROWS_PT = NP // NS
ZR = 64
RING = 4
LEAD = 2
UNROLL = 2


def _nsum_body(xh_hbm, src_hbm, dst_hbm, out_hbm, src_v, dst_v, rows, zbuf, acc_sh, isem, g, s):
    cid = lax.axis_index("c")
    sid = lax.axis_index("s")

    ic1 = pltpu.async_copy(src_hbm.at[sid], src_v, isem)
    ic2 = pltpu.async_copy(dst_hbm.at[sid], dst_v, isem)

    def zrow(i, _):
        def zcol(j, _):
            zbuf[i, pl.ds(j * 16, 16)] = jnp.zeros((16,), jnp.float32)
            return 0
        return lax.fori_loop(0, DH // 16, zcol, 0)
    lax.fori_loop(0, ZR, zrow, 0)
    for q in range(ROWS_PT // ZR):
        pltpu.sync_copy(zbuf, acc_sh.at[pl.ds(sid * ROWS_PT + q * ZR, ZR)])
    ic1.wait()
    ic2.wait()
    plsc.subcore_barrier()

    xt = xh_hbm.at[cid]

    def step(k, _):
        for j in range(UNROLL):
            c = k * UNROLL + j

            @pl.when(c < NCHUNK)
            def _():
                b = lax.rem(c, RING)
                pltpu.async_copy(xt.at[src_v.at[c]], rows.at[b], g.at[b])

            @pl.when(jnp.logical_and(c >= LEAD, c < NCHUNK + LEAD))
            def _():
                kk = c - LEAD
                b2 = lax.rem(kk, RING)
                pltpu.make_async_copy(xt.at[src_v.at[kk]], rows.at[b2], g.at[b2]).wait()

                @pl.when(kk >= 1)
                def _():
                    b1 = lax.rem(kk - 1, RING)
                    pltpu.make_async_copy(
                        rows.at[b1], acc_sh.at[dst_v.at[kk - 1]], s.at[b1]).wait()
                pltpu.async_copy(rows.at[b2], acc_sh.at[dst_v.at[kk]], s.at[b2], add=True)
        return 0
    lax.fori_loop(0, (NCHUNK + LEAD + UNROLL - 1) // UNROLL, step, 0)
    bl = (NCHUNK - 1) % RING
    pltpu.make_async_copy(rows.at[bl], acc_sh.at[dst_v.at[NCHUNK - 1]], s.at[bl]).wait()
    plsc.subcore_barrier()

    r0 = sid * ROWS_PT
    pltpu.sync_copy(acc_sh.at[pl.ds(r0, ROWS_PT)], out_hbm.at[cid].at[pl.ds(r0, ROWS_PT)])


_neighbor_sum_sc = pl.kernel(
    _nsum_body,
    out_type=jax.ShapeDtypeStruct((NC, NP, DH), jnp.float32),
    mesh=plsc.VectorSubcoreMesh(core_axis_name="c", subcore_axis_name="s",
                                num_cores=NC, num_subcores=NS),
    compiler_params=pltpu.CompilerParams(use_tc_tiling_on_sc=False),
    scratch_types=[
        pltpu.VMEM((NCHUNK, CH), jnp.int32),
        pltpu.VMEM((NCHUNK, CH), jnp.int32),
        pltpu.VMEM((RING, CH, DH), jnp.float32),
        pltpu.VMEM((ZR, DH), jnp.float32),
        pltpu.VMEM_SHARED((NP, DH), jnp.float32),
        pltpu.SemaphoreType.DMA,
        pltpu.SemaphoreType.DMA((RING,)),
        pltpu.SemaphoreType.DMA((RING,)),
    ],
)


BN = 1000


def _mlp_block(eps_ref, x_ref, p_ref, W1_ref, b1_ref, W2_ref, b2_ref, o_ref):
    xb = jnp.concatenate([x_ref[0], x_ref[1]], axis=-1)
    pb = jnp.concatenate([p_ref[0], p_ref[1]], axis=-1)
    h = (1.0 + eps_ref[0, 0]) * xb + pb
    h = jnp.maximum(jnp.dot(h, W1_ref[...], preferred_element_type=jnp.float32)
                    + b1_ref[...], 0.0)
    h = jnp.dot(h, W2_ref[...], preferred_element_type=jnp.float32) + b2_ref[...]
    h = jnp.maximum(h, 0.0)
    o_ref[0] = h[:, :DH]
    o_ref[1] = h[:, DH:]


def _cls_block(eps_ref, x_ref, p_ref, Wc_ref, bc_ref, o_ref):
    xb = jnp.concatenate([x_ref[0], x_ref[1]], axis=-1)
    pb = jnp.concatenate([p_ref[0], p_ref[1]], axis=-1)
    h = (1.0 + eps_ref[0, 0]) * xb + pb
    o_ref[...] = jnp.dot(h, Wc_ref[...], preferred_element_type=jnp.float32) + bc_ref[...]


def _gin_mlp_tc(eps, x, p, W1, b1, W2, b2):
    return pl.pallas_call(
        _mlp_block,
        grid=(N // BN,),
        in_specs=[
            pl.BlockSpec(memory_space=pltpu.SMEM),
            pl.BlockSpec((NC, BN, DH), lambda i: (0, i, 0)),
            pl.BlockSpec((NC, BN, DH), lambda i: (0, i, 0)),
            pl.BlockSpec((D, H), lambda i: (0, 0)),
            pl.BlockSpec((1, H), lambda i: (0, 0)),
            pl.BlockSpec((H, H), lambda i: (0, 0)),
            pl.BlockSpec((1, H), lambda i: (0, 0)),
        ],
        out_specs=pl.BlockSpec((NC, BN, DH), lambda i: (0, i, 0)),
        out_shape=jax.ShapeDtypeStruct((NC, N, DH), jnp.float32),
    )(eps, x, p, W1, b1, W2, b2)


def _gin_cls_tc(eps, x, p, Wc, bc):
    return pl.pallas_call(
        _cls_block,
        grid=(N // BN,),
        in_specs=[
            pl.BlockSpec(memory_space=pltpu.SMEM),
            pl.BlockSpec((NC, BN, DH), lambda i: (0, i, 0)),
            pl.BlockSpec((NC, BN, DH), lambda i: (0, i, 0)),
            pl.BlockSpec((H, C), lambda i: (0, 0)),
            pl.BlockSpec((1, C), lambda i: (0, 0)),
        ],
        out_specs=pl.BlockSpec((BN, C), lambda i: (i, 0)),
        out_shape=jax.ShapeDtypeStruct((N, C), jnp.float32),
    )(eps, x, p, Wc, bc)


def kernel(features, edge_index, eps0, W1_0, b1_0, W2_0, b2_0,
           eps1, W1_1, b1_1, W2_1, b2_1, epsc, Wc, bc):
    padi = jnp.arange(EPAD, dtype=jnp.int32)
    src = jnp.concatenate([edge_index[0], padi % N]).reshape(NS, NCHUNK, CH)
    dst = jnp.concatenate([edge_index[1], N + padi % (NP - N)]).reshape(NS, NCHUNK, CH)
    src, dst = jax.lax.optimization_barrier((src, dst))
    e0 = eps0.reshape(1, 1)
    e1 = eps1.reshape(1, 1)
    ec = epsc.reshape(1, 1)

    x = jnp.stack([features[:, :DH], features[:, DH:]])
    p = _neighbor_sum_sc(x, src, dst)
    x = _gin_mlp_tc(e0, x, p, W1_0, b1_0.reshape(1, H), W2_0, b2_0.reshape(1, H))
    p = _neighbor_sum_sc(x, src, dst)
    x = _gin_mlp_tc(e1, x, p, W1_1, b1_1.reshape(1, H), W2_1, b2_1.reshape(1, H))
    p = _neighbor_sum_sc(x, src, dst)
    return _gin_cls_tc(ec, x, p, Wc, bc.reshape(1, C))


# speedup vs baseline: 1.2041x; 1.2041x over previous
import jax
import jax.numpy as jnp
from jax import lax
from jax.experimental import pallas as pl
from jax.experimental.pallas import tpu as pltpu
from jax.experimental.pallas import tpu_sc as plsc

N = 10000
E = 320000
D = 128
H = 128
C = 40
DH = D // 2

NC = 2
NS = 16
EPT = E // NS
CH = 80
NCHUNK = EPT // CH
NP = 10240
ROWS_PT = NP // NS
ZR = 64
RING = 8
LEAD = 4
UNROLL = 2


def _nsum_body(x2_hbm, src_hbm, dst_hbm, out_hbm, src_v, dst_v, rows, zbuf, acc_sh, isem, g, s):
    cid = lax.axis_index("c")
    sid = lax.axis_index("s")

    ic1 = pltpu.async_copy(src_hbm.at[cid, sid], src_v, isem)
    ic2 = pltpu.async_copy(dst_hbm.at[sid], dst_v, isem)

    def zrow(i, _):
        def zcol(j, _):
            zbuf[i, pl.ds(j * 16, 16)] = jnp.zeros((16,), jnp.float32)
            return 0
        return lax.fori_loop(0, DH // 16, zcol, 0)
    lax.fori_loop(0, ZR, zrow, 0)
    for q in range(ROWS_PT // ZR):
        pltpu.sync_copy(zbuf, acc_sh.at[pl.ds(sid * ROWS_PT + q * ZR, ZR)])
    ic1.wait()
    ic2.wait()
    plsc.subcore_barrier()

    def step(k, _):
        for j in range(UNROLL):
            c = k * UNROLL + j

            @pl.when(c < NCHUNK)
            def _():
                b = lax.rem(c, RING)
                pltpu.async_copy(x2_hbm.at[src_v.at[c]], rows.at[b], g.at[b])

            @pl.when(jnp.logical_and(c >= LEAD, c < NCHUNK + LEAD))
            def _():
                kk = c - LEAD
                b2 = lax.rem(kk, RING)
                pltpu.make_async_copy(x2_hbm.at[src_v.at[kk]], rows.at[b2], g.at[b2]).wait()

                @pl.when(kk >= 1)
                def _():
                    b1 = lax.rem(kk - 1, RING)
                    pltpu.make_async_copy(
                        rows.at[b1], acc_sh.at[dst_v.at[kk - 1]], s.at[b1]).wait()
                pltpu.async_copy(rows.at[b2], acc_sh.at[dst_v.at[kk]], s.at[b2], add=True)
        return 0
    lax.fori_loop(0, (NCHUNK + LEAD) // UNROLL, step, 0)
    bl = (NCHUNK - 1) % RING
    pltpu.make_async_copy(rows.at[bl], acc_sh.at[dst_v.at[NCHUNK - 1]], s.at[bl]).wait()
    plsc.subcore_barrier()

    r0 = sid * ROWS_PT
    acc_slice = acc_sh.at[pl.ds(r0, ROWS_PT)]

    @pl.when(cid == 0)
    def _():
        pltpu.sync_copy(acc_slice, out_hbm.at[pl.ds(r0, ROWS_PT), pl.ds(0, DH)])

    @pl.when(cid == 1)
    def _():
        pltpu.sync_copy(acc_slice, out_hbm.at[pl.ds(r0, ROWS_PT), pl.ds(DH, DH)])


_neighbor_sum_sc = pl.kernel(
    _nsum_body,
    out_type=jax.ShapeDtypeStruct((NP, D), jnp.float32),
    mesh=plsc.VectorSubcoreMesh(core_axis_name="c", subcore_axis_name="s",
                                num_cores=NC, num_subcores=NS),
    compiler_params=pltpu.CompilerParams(use_tc_tiling_on_sc=False),
    scratch_types=[
        pltpu.VMEM((NCHUNK, CH), jnp.int32),
        pltpu.VMEM((NCHUNK, CH), jnp.int32),
        pltpu.VMEM((RING, CH, DH), jnp.float32),
        pltpu.VMEM((ZR, DH), jnp.float32),
        pltpu.VMEM_SHARED((NP, DH), jnp.float32),
        pltpu.SemaphoreType.DMA,
        pltpu.SemaphoreType.DMA((RING,)),
        pltpu.SemaphoreType.DMA((RING,)),
    ],
)


BN = 1000


def _mlp_block(eps_ref, x_ref, p_ref, W1_ref, b1_ref, W2_ref, b2_ref, o_ref):
    h = (1.0 + eps_ref[0, 0]) * x_ref[...] + p_ref[...]
    h = jnp.maximum(jnp.dot(h, W1_ref[...], preferred_element_type=jnp.float32)
                    + b1_ref[...], 0.0)
    h = jnp.dot(h, W2_ref[...], preferred_element_type=jnp.float32) + b2_ref[...]
    o_ref[...] = jnp.maximum(h, 0.0)


def _cls_block(eps_ref, x_ref, p_ref, Wc_ref, bc_ref, o_ref):
    h = (1.0 + eps_ref[0, 0]) * x_ref[...] + p_ref[...]
    o_ref[...] = jnp.dot(h, Wc_ref[...], preferred_element_type=jnp.float32) + bc_ref[...]


def _gin_mlp_tc(eps, x, p, W1, b1, W2, b2):
    return pl.pallas_call(
        _mlp_block,
        grid=(N // BN,),
        in_specs=[
            pl.BlockSpec(memory_space=pltpu.SMEM),
            pl.BlockSpec((BN, D), lambda i: (i, 0)),
            pl.BlockSpec((BN, D), lambda i: (i, 0)),
            pl.BlockSpec((D, H), lambda i: (0, 0)),
            pl.BlockSpec((1, H), lambda i: (0, 0)),
            pl.BlockSpec((H, H), lambda i: (0, 0)),
            pl.BlockSpec((1, H), lambda i: (0, 0)),
        ],
        out_specs=pl.BlockSpec((BN, H), lambda i: (i, 0)),
        out_shape=jax.ShapeDtypeStruct((N, H), jnp.float32),
    )(eps, x, p, W1, b1, W2, b2)


def _gin_cls_tc(eps, x, p, Wc, bc):
    return pl.pallas_call(
        _cls_block,
        grid=(N // BN,),
        in_specs=[
            pl.BlockSpec(memory_space=pltpu.SMEM),
            pl.BlockSpec((BN, H), lambda i: (i, 0)),
            pl.BlockSpec((BN, H), lambda i: (i, 0)),
            pl.BlockSpec((H, C), lambda i: (0, 0)),
            pl.BlockSpec((1, C), lambda i: (0, 0)),
        ],
        out_specs=pl.BlockSpec((BN, C), lambda i: (i, 0)),
        out_shape=jax.ShapeDtypeStruct((N, C), jnp.float32),
    )(eps, x, p, Wc, bc)


def kernel(features, edge_index, eps0, W1_0, b1_0, W2_0, b2_0,
           eps1, W1_1, b1_1, W2_1, b2_1, epsc, Wc, bc):
    srcp = edge_index[0].reshape(NS, NCHUNK, CH)
    src2 = jnp.stack([srcp * 2, srcp * 2 + 1])
    dst = edge_index[1].reshape(NS, NCHUNK, CH)
    e0 = eps0.reshape(1, 1)
    e1 = eps1.reshape(1, 1)
    ec = epsc.reshape(1, 1)

    p = _neighbor_sum_sc(features.reshape(2 * N, DH), src2, dst)
    x = _gin_mlp_tc(e0, features, p, W1_0, b1_0.reshape(1, H), W2_0, b2_0.reshape(1, H))
    p = _neighbor_sum_sc(x.reshape(2 * N, DH), src2, dst)
    x = _gin_mlp_tc(e1, x, p, W1_1, b1_1.reshape(1, H), W2_1, b2_1.reshape(1, H))
    p = _neighbor_sum_sc(x.reshape(2 * N, DH), src2, dst)
    return _gin_cls_tc(ec, x, p, Wc, bc.reshape(1, C))


# speedup vs baseline: 1.2360x; 1.0265x over previous
import jax
import jax.numpy as jnp
from jax import lax
from jax.experimental import pallas as pl
from jax.experimental.pallas import tpu as pltpu
from jax.experimental.pallas import tpu_sc as plsc

N = 10000
E = 320000
D = 128
H = 128
C = 40
DH = D // 2

NC = 2
NS = 16
EPT = E // NS
CH = 80
NCHUNK = EPT // CH
NP = 10240
ROWS_PT = NP // NS
ZR = 64
RING = 8
LEAD = 4
UNROLL = 2


def _nsum_body(x2_hbm, src_hbm, dst_hbm, out_hbm, src_v, dst_v, rows, zbuf, acc_sh, isem, g, s):
    cid = lax.axis_index("c")
    sid = lax.axis_index("s")

    ic1 = pltpu.async_copy(src_hbm.at[cid, sid], src_v, isem)
    ic2 = pltpu.async_copy(dst_hbm.at[sid], dst_v, isem)

    def zrow(i, _):
        def zcol(j, _):
            zbuf[i, pl.ds(j * 16, 16)] = jnp.zeros((16,), jnp.float32)
            return 0
        return lax.fori_loop(0, DH // 16, zcol, 0)
    lax.fori_loop(0, ZR, zrow, 0)
    for q in range(ROWS_PT // ZR):
        pltpu.sync_copy(zbuf, acc_sh.at[pl.ds(sid * ROWS_PT + q * ZR, ZR)])
    ic1.wait()
    ic2.wait()
    plsc.subcore_barrier()

    def step(k, _):
        for j in range(UNROLL):
            c = k * UNROLL + j

            @pl.when(c < NCHUNK)
            def _():
                b = lax.rem(c, RING)
                pltpu.async_copy(x2_hbm.at[src_v.at[c]], rows.at[b], g.at[b])

            @pl.when(jnp.logical_and(c >= LEAD, c < NCHUNK + LEAD))
            def _():
                kk = c - LEAD
                b2 = lax.rem(kk, RING)
                pltpu.make_async_copy(x2_hbm.at[src_v.at[kk]], rows.at[b2], g.at[b2]).wait()

                @pl.when(kk >= 1)
                def _():
                    b1 = lax.rem(kk - 1, RING)
                    pltpu.make_async_copy(
                        rows.at[b1], acc_sh.at[dst_v.at[kk - 1]], s.at[b1]).wait()
                pltpu.async_copy(rows.at[b2], acc_sh.at[dst_v.at[kk]], s.at[b2], add=True)
        return 0
    lax.fori_loop(0, (NCHUNK + LEAD) // UNROLL, step, 0)
    bl = (NCHUNK - 1) % RING
    pltpu.make_async_copy(rows.at[bl], acc_sh.at[dst_v.at[NCHUNK - 1]], s.at[bl]).wait()
    plsc.subcore_barrier()

    r0 = sid * ROWS_PT
    acc_slice = acc_sh.at[pl.ds(r0, ROWS_PT)]

    @pl.when(cid == 0)
    def _():
        pltpu.sync_copy(acc_slice, out_hbm.at[pl.ds(r0, ROWS_PT), pl.ds(0, DH)])

    @pl.when(cid == 1)
    def _():
        pltpu.sync_copy(acc_slice, out_hbm.at[pl.ds(r0, ROWS_PT), pl.ds(DH, DH)])


_neighbor_sum_sc = pl.kernel(
    _nsum_body,
    out_type=jax.ShapeDtypeStruct((NP, D), jnp.float32),
    mesh=plsc.VectorSubcoreMesh(core_axis_name="c", subcore_axis_name="s",
                                num_cores=NC, num_subcores=NS),
    compiler_params=pltpu.CompilerParams(use_tc_tiling_on_sc=False),
    scratch_types=[
        pltpu.VMEM((NCHUNK, CH), jnp.int32),
        pltpu.VMEM((NCHUNK, CH), jnp.int32),
        pltpu.VMEM((RING, CH, DH), jnp.float32),
        pltpu.VMEM((ZR, DH), jnp.float32),
        pltpu.VMEM_SHARED((NP, DH), jnp.float32),
        pltpu.SemaphoreType.DMA,
        pltpu.SemaphoreType.DMA((RING,)),
        pltpu.SemaphoreType.DMA((RING,)),
    ],
)


BN = 2000


def _mlp_block(eps_ref, x_ref, p_ref, W1_ref, b1_ref, W2_ref, b2_ref, o_ref):
    h = (1.0 + eps_ref[0, 0]) * x_ref[...] + p_ref[...]
    h = jnp.maximum(jnp.dot(h, W1_ref[...], preferred_element_type=jnp.float32)
                    + b1_ref[...], 0.0)
    h = jnp.dot(h, W2_ref[...], preferred_element_type=jnp.float32) + b2_ref[...]
    o_ref[...] = jnp.maximum(h, 0.0)


def _cls_block(eps_ref, x_ref, p_ref, Wc_ref, bc_ref, o_ref):
    h = (1.0 + eps_ref[0, 0]) * x_ref[...] + p_ref[...]
    o_ref[...] = jnp.dot(h, Wc_ref[...], preferred_element_type=jnp.float32) + bc_ref[...]


def _gin_mlp_tc(eps, x, p, W1, b1, W2, b2):
    return pl.pallas_call(
        _mlp_block,
        grid=(N // BN,),
        in_specs=[
            pl.BlockSpec(memory_space=pltpu.SMEM),
            pl.BlockSpec((BN, D), lambda i: (i, 0)),
            pl.BlockSpec((BN, D), lambda i: (i, 0)),
            pl.BlockSpec((D, H), lambda i: (0, 0)),
            pl.BlockSpec((1, H), lambda i: (0, 0)),
            pl.BlockSpec((H, H), lambda i: (0, 0)),
            pl.BlockSpec((1, H), lambda i: (0, 0)),
        ],
        out_specs=pl.BlockSpec((BN, H), lambda i: (i, 0)),
        out_shape=jax.ShapeDtypeStruct((N, H), jnp.float32),
    )(eps, x, p, W1, b1, W2, b2)


def _gin_cls_tc(eps, x, p, Wc, bc):
    return pl.pallas_call(
        _cls_block,
        grid=(N // BN,),
        in_specs=[
            pl.BlockSpec(memory_space=pltpu.SMEM),
            pl.BlockSpec((BN, H), lambda i: (i, 0)),
            pl.BlockSpec((BN, H), lambda i: (i, 0)),
            pl.BlockSpec((H, C), lambda i: (0, 0)),
            pl.BlockSpec((1, C), lambda i: (0, 0)),
        ],
        out_specs=pl.BlockSpec((BN, C), lambda i: (i, 0)),
        out_shape=jax.ShapeDtypeStruct((N, C), jnp.float32),
    )(eps, x, p, Wc, bc)


def kernel(features, edge_index, eps0, W1_0, b1_0, W2_0, b2_0,
           eps1, W1_1, b1_1, W2_1, b2_1, epsc, Wc, bc):
    srcp = edge_index[0].reshape(NS, NCHUNK, CH)
    src2 = jnp.stack([srcp * 2, srcp * 2 + 1])
    dst = edge_index[1].reshape(NS, NCHUNK, CH)
    e0 = eps0.reshape(1, 1)
    e1 = eps1.reshape(1, 1)
    ec = epsc.reshape(1, 1)

    p = _neighbor_sum_sc(features.reshape(2 * N, DH), src2, dst)
    x = _gin_mlp_tc(e0, features, p, W1_0, b1_0.reshape(1, H), W2_0, b2_0.reshape(1, H))
    p = _neighbor_sum_sc(x.reshape(2 * N, DH), src2, dst)
    x = _gin_mlp_tc(e1, x, p, W1_1, b1_1.reshape(1, H), W2_1, b2_1.reshape(1, H))
    p = _neighbor_sum_sc(x.reshape(2 * N, DH), src2, dst)
    return _gin_cls_tc(ec, x, p, Wc, bc.reshape(1, C))
